# baseline (speedup 1.0000x reference)
import functools

import jax
import jax.numpy as jnp
from jax import lax
from jax.experimental import pallas as pl
from jax.experimental.pallas import tpu as pltpu
from jax.experimental.pallas import tpu_sc as plsc

NN = 10000
EE = 320000
H = 8
OUT = 16
HID = 128
HXA_W = 144

NTILES = 32
CHG = 64
CHS = 128
ZCH = 80
NZ = NN // ZCH

_MM = dict(preferred_element_type=jnp.float32)


def _tc1_body(sf, nf, w1, b1, w2, b2, gw, a_s, a_d, so_ref, hxa_ref, adst_ref):
    h1 = jnp.maximum(jnp.dot(sf[...], w1[...], **_MM) + b1[...], 0.0)
    so_ref[...] = jnp.maximum(jnp.dot(h1, w2[...], **_MM) + b2[...], 0.0)
    hx = jnp.dot(nf[...], gw[...], **_MM)
    hxa_ref[:, 0:128] = hx
    hxa_ref[:, 128:136] = jnp.dot(hx, a_s[...], **_MM)
    hxa_ref[:, 136:144] = jnp.zeros_like(hxa_ref[:, 136:144])
    adst_ref[:, 0:8] = jnp.dot(hx, a_d[...], **_MM)
    adst_ref[:, 8:16] = jnp.zeros_like(adst_ref[:, 8:16])


def _tc1(sf, nf, w1, b1, w2, b2, gw, a_s, a_d):
    B = 1000
    g = NN // B
    row = lambda i: (i, 0)
    full = lambda i: (0, 0)
    return pl.pallas_call(
        _tc1_body,
        grid=(g,),
        in_specs=[
            pl.BlockSpec((B, 65), row),
            pl.BlockSpec((B, HID), row),
            pl.BlockSpec((65, HID), full),
            pl.BlockSpec((1, HID), full),
            pl.BlockSpec((HID, HID), full),
            pl.BlockSpec((1, HID), full),
            pl.BlockSpec((HID, HID), full),
            pl.BlockSpec((HID, H), full),
            pl.BlockSpec((HID, H), full),
        ],
        out_specs=[
            pl.BlockSpec((B, HID), row),
            pl.BlockSpec((B, HXA_W), row),
            pl.BlockSpec((B, 16), row),
        ],
        out_shape=[
            jax.ShapeDtypeStruct((NN, HID), jnp.float32),
            jax.ShapeDtypeStruct((NN, HXA_W), jnp.float32),
            jax.ShapeDtypeStruct((NN, 16), jnp.float32),
        ],
    )(sf, nf, w1, b1, w2, b2, gw, a_s, a_d)


def _zero_accum(buf_v, accum, sid, width, zch):
    nz = NN // zch

    def zrow(r, _):
        for kk in range(width // 16):
            buf_v[r, pl.ds(kk * 16, 16)] = jnp.zeros((16,), jnp.float32)
        return 0

    lax.fori_loop(0, zch, zrow, 0)

    def zac(k, _):
        idx = sid + k * 16

        @pl.when(idx < nz)
        def _():
            pltpu.sync_copy(buf_v.at[pl.ds(0, zch)],
                            accum.at[pl.ds(idx * zch, zch)])

        return 0

    lax.fori_loop(0, (nz + 15) // 16, zac, 0)


def _writeback(accum, out_hbm, cid, sid):
    def wb(k, _):
        idx = sid + k * 16

        @pl.when(idx < NZ)
        def _():
            pltpu.sync_copy(accum.at[pl.ds(idx * ZCH, ZCH)],
                            out_hbm.at[cid, pl.ds(idx * ZCH, ZCH)])

        return 0

    lax.fori_loop(0, (NZ + 15) // 16, wb, 0)


def _writeback_split(accum, out1_hbm, out2_hbm, cid, sid):
    def wb(k, _):
        idx = sid + k * 16

        @pl.when(idx < NZ)
        def _():
            pltpu.sync_copy(accum.at[pl.ds(idx * ZCH, ZCH), pl.ds(0, HID)],
                            out1_hbm.at[cid, pl.ds(idx * ZCH, ZCH)])
            pltpu.sync_copy(accum.at[pl.ds(idx * ZCH, ZCH), pl.ds(HID, 16)],
                            out2_hbm.at[cid, pl.ds(idx * ZCH, ZCH)])

        return 0

    lax.fori_loop(0, (NZ + 15) // 16, wb, 0)


def _copy_dsc(ei_v, dsc_v, ch):
    for kk in range(ch // 16):
        dsc_v[pl.ds(kk * 16, 16)] = ei_v[1, pl.ds(kk * 16, 16)]


def _gat_edges_body(hxa_hbm, adst_hbm, ei_hbm, out1_hbm, out2_hbm, accum,
                    ei_v0, ei_v1, ei_v2, hxa_v0, hxa_v1, hxa_v2,
                    adst_v0, adst_v1, adst_v2, dsc_v0, dsc_v1, dsc_v2,
                    sem_i0, sem_i1, sem_i2, sem_g0, sem_g1, sem_g2,
                    sem_a0, sem_a1, sem_a2, sem_s0, sem_s1, sem_s2):
    cid = lax.axis_index("c")
    sid = lax.axis_index("s")
    wid = sid * 2 + cid
    nchunk = EE // CHG
    njs = (nchunk + NTILES - 1) // NTILES
    nj = (nchunk + NTILES - 1 - wid) // NTILES

    ei_v = (ei_v0, ei_v1, ei_v2)
    hxa_v = (hxa_v0, hxa_v1, hxa_v2)
    adst_v = (adst_v0, adst_v1, adst_v2)
    dsc_v = (dsc_v0, dsc_v1, dsc_v2)
    sem_i = (sem_i0, sem_i1, sem_i2)
    sem_g = (sem_g0, sem_g1, sem_g2)
    sem_a = (sem_a0, sem_a1, sem_a2)
    sem_s = (sem_s0, sem_s1, sem_s2)

    _zero_accum(hxa_v0, accum, sid, HXA_W, 40)
    plsc.subcore_barrier()

    def idx_args(j, s):
        return ei_hbm.at[:, pl.ds((wid + j * NTILES) * CHG, CHG)], ei_v[s], sem_i[s]

    def gather_args(s):
        return ((hxa_hbm.at[ei_v[s].at[0]], hxa_v[s], sem_g[s]),
                (adst_hbm.at[ei_v[s].at[1]], adst_v[s], sem_a[s]))

    def scatter_args(s):
        return hxa_v[s], accum.at[dsc_v[s]], sem_s[s]

    def compute(s):
        lane = lax.iota(jnp.int32, 16)
        mlo = jnp.where(lane < 8, 1.0, 0.0)
        dg1 = jnp.where(lane == 8, 1.0, 0.0)

        @plsc.parallel_loop(0, CHG, unroll=8)
        def _(c):
            av = hxa_v[s][c, pl.ds(128, 16)] + adst_v[s][c, pl.ds(0, 16)]
            al = jnp.where(av > 0.0, av, av * 0.2)
            tail = jnp.exp(al) * mlo + dg1
            for h in range(H):
                hxa_v[s][c, pl.ds(h * 16, 16)] = (
                    hxa_v[s][c, pl.ds(h * 16, 16)] * tail[h])
            hxa_v[s][c, pl.ds(128, 16)] = tail

    pltpu.async_copy(*idx_args(0, 0)).wait()
    for a in gather_args(0):
        pltpu.async_copy(*a)
    pltpu.async_copy(*idx_args(1, 1))
    pltpu.async_copy(*idx_args(2, 2))

    def pipe(jj, _):
        for b in (0, 1, 2):
            j = jj * 3 + b
            b1 = (b + 1) % 3

            @pl.when(j + 1 < nj)
            def _():
                pltpu.make_async_copy(*idx_args(j + 1, b1)).wait()

                @pl.when(j >= 2)
                def _():
                    pltpu.make_async_copy(*scatter_args(b1)).wait()

                for a in gather_args(b1):
                    pltpu.async_copy(*a)

            @pl.when(j < nj)
            def _():
                for a in gather_args(b):
                    pltpu.make_async_copy(*a).wait()
                _copy_dsc(ei_v[b], dsc_v[b], CHG)

            @pl.when(j + 3 < nj)
            def _():
                pltpu.async_copy(*idx_args(j + 3, b))

            @pl.when(j < nj)
            def _():
                compute(b)
                pltpu.async_copy(*scatter_args(b), add=True)

        return 0

    lax.fori_loop(0, (njs + 2) // 3, pipe, 0)
    pltpu.make_async_copy(*scatter_args(0)).wait()
    pltpu.make_async_copy(*scatter_args(1)).wait()
    pltpu.make_async_copy(*scatter_args(2)).wait()
    plsc.subcore_barrier()
    _writeback_split(accum, out1_hbm, out2_hbm, cid, sid)


def _gat_edges(hxa, adst, ei):
    mesh = plsc.VectorSubcoreMesh(core_axis_name="c", subcore_axis_name="s", num_cores=2, num_subcores=16)
    return pl.kernel(
        _gat_edges_body,
        out_type=[jax.ShapeDtypeStruct((2, NN, HID), jnp.float32),
                  jax.ShapeDtypeStruct((2, NN, 16), jnp.float32)],
        mesh=mesh,
        compiler_params=pltpu.CompilerParams(use_tc_tiling_on_sc=False),
        scratch_types=[
            pltpu.VMEM_SHARED((NN, HXA_W), jnp.float32),
            pltpu.VMEM((2, CHG), jnp.int32),
            pltpu.VMEM((2, CHG), jnp.int32),
            pltpu.VMEM((2, CHG), jnp.int32),
            pltpu.VMEM((CHG, HXA_W), jnp.float32),
            pltpu.VMEM((CHG, HXA_W), jnp.float32),
            pltpu.VMEM((CHG, HXA_W), jnp.float32),
            pltpu.VMEM((CHG, 16), jnp.float32),
            pltpu.VMEM((CHG, 16), jnp.float32),
            pltpu.VMEM((CHG, 16), jnp.float32),
            pltpu.VMEM((CHG,), jnp.int32),
            pltpu.VMEM((CHG,), jnp.int32),
            pltpu.VMEM((CHG,), jnp.int32),
            pltpu.SemaphoreType.DMA,
            pltpu.SemaphoreType.DMA,
            pltpu.SemaphoreType.DMA,
            pltpu.SemaphoreType.DMA,
            pltpu.SemaphoreType.DMA,
            pltpu.SemaphoreType.DMA,
            pltpu.SemaphoreType.DMA,
            pltpu.SemaphoreType.DMA,
            pltpu.SemaphoreType.DMA,
            pltpu.SemaphoreType.DMA,
            pltpu.SemaphoreType.DMA,
            pltpu.SemaphoreType.DMA,
        ],
    )(hxa, adst, ei)


def _tc2_body(pn0v, pn1v, pt0v, pt1v, hxa, adst, gb, rmat, wr,
              x_ref, xr_ref, deg_ref):
    pn0 = pn0v[0]
    pn1 = pn1v[0]
    pt0 = pt0v[0]
    pt1 = pt1v[0]
    a_s = hxa[:, 128:136]
    a_d = adst[:, 0:8]
    t = a_s + a_d
    al = jnp.where(t > 0.0, t, t * 0.2)
    exl = jnp.exp(al)
    denom = pt0[:, 0:8] + pt1[:, 0:8] + exl
    exl_rep = jnp.dot(exl, rmat[...], **_MM)
    den_rep = jnp.dot(denom, rmat[...], **_MM)
    msum = pn0[...] + pn1[...] + hxa[:, 0:128] * exl_rep
    gat = msum / (den_rep + 1e-16)
    xx = gat + gb[...]
    x = jnp.where(xx > 0.0, xx, xx * 0.2)
    x_ref[...] = x
    xr_ref[...] = jnp.dot(x, wr[...], **_MM)
    deg_ref[...] = pt0[:, 8:9] + pt1[:, 8:9]


def _tc2(pn, pt, hxa, adst, gb, rmat, wr):
    B = 1000
    g = NN // B
    row = lambda i: (i, 0)
    full = lambda i: (0, 0)
    return pl.pallas_call(
        _tc2_body,
        grid=(g,),
        in_specs=[
            pl.BlockSpec((1, B, HID), lambda i: (0, i, 0)),
            pl.BlockSpec((1, B, HID), lambda i: (1, i, 0)),
            pl.BlockSpec((1, B, 16), lambda i: (0, i, 0)),
            pl.BlockSpec((1, B, 16), lambda i: (1, i, 0)),
            pl.BlockSpec((B, HXA_W), row),
            pl.BlockSpec((B, 16), row),
            pl.BlockSpec((1, HID), full),
            pl.BlockSpec((H, HID), full),
            pl.BlockSpec((HID, HID), full),
        ],
        out_specs=[
            pl.BlockSpec((B, HID), row),
            pl.BlockSpec((B, HID), row),
            pl.BlockSpec((B, 1), row),
        ],
        out_shape=[
            jax.ShapeDtypeStruct((NN, HID), jnp.float32),
            jax.ShapeDtypeStruct((NN, HID), jnp.float32),
            jax.ShapeDtypeStruct((NN, 1), jnp.float32),
        ],
    )(pn, pn, pt, pt, hxa, adst, gb, rmat, wr)


def _sage_edges_body(x_hbm, ei_hbm, out_hbm, accum,
                     ei_v0, ei_v1, x_v0, x_v1, dsc_v0, dsc_v1,
                     sem_i0, sem_i1, sem_g0, sem_g1, sem_s0, sem_s1):
    cid = lax.axis_index("c")
    sid = lax.axis_index("s")
    wid = sid * 2 + cid
    nchunk = EE // CHS
    njs = (nchunk + NTILES - 1) // NTILES
    nj = (nchunk + NTILES - 1 - wid) // NTILES

    ei_v = (ei_v0, ei_v1)
    x_v = (x_v0, x_v1)
    dsc_v = (dsc_v0, dsc_v1)
    sem_i = (sem_i0, sem_i1)
    sem_g = (sem_g0, sem_g1)
    sem_s = (sem_s0, sem_s1)

    _zero_accum(x_v0, accum, sid, HID, ZCH)
    plsc.subcore_barrier()

    def idx_args(j, s):
        return ei_hbm.at[:, pl.ds((wid + j * NTILES) * CHS, CHS)], ei_v[s], sem_i[s]

    def gather_args(s):
        return x_hbm.at[ei_v[s].at[0]], x_v[s], sem_g[s]

    def scatter_args(s):
        return x_v[s], accum.at[dsc_v[s]], sem_s[s]

    pltpu.async_copy(*idx_args(0, 0)).wait()
    pltpu.async_copy(*gather_args(0))
    pltpu.async_copy(*idx_args(1, 1))

    def pipe(jj, _):
        for b in (0, 1):
            j = jj * 2 + b
            nb = 1 - b

            @pl.when(j + 1 < nj)
            def _():
                pltpu.make_async_copy(*idx_args(j + 1, nb)).wait()

                @pl.when(j >= 1)
                def _():
                    pltpu.make_async_copy(*scatter_args(nb)).wait()

                pltpu.async_copy(*gather_args(nb))

            @pl.when(j < nj)
            def _():
                pltpu.make_async_copy(*gather_args(b)).wait()
                _copy_dsc(ei_v[b], dsc_v[b], CHS)

            @pl.when(j + 2 < nj)
            def _():
                pltpu.async_copy(*idx_args(j + 2, b))

            @pl.when(j < nj)
            def _():
                pltpu.async_copy(*scatter_args(b), add=True)

        return 0

    lax.fori_loop(0, (njs + 1) // 2, pipe, 0)
    pltpu.make_async_copy(*scatter_args(0)).wait()
    pltpu.make_async_copy(*scatter_args(1)).wait()
    plsc.subcore_barrier()
    _writeback(accum, out_hbm, cid, sid)


def _sage_edges(x, ei):
    mesh = plsc.VectorSubcoreMesh(core_axis_name="c", subcore_axis_name="s", num_cores=2, num_subcores=16)
    return pl.kernel(
        _sage_edges_body,
        out_type=jax.ShapeDtypeStruct((2, NN, HID), jnp.float32),
        mesh=mesh,
        compiler_params=pltpu.CompilerParams(use_tc_tiling_on_sc=False),
        scratch_types=[
            pltpu.VMEM_SHARED((NN, HID), jnp.float32),
            pltpu.VMEM((2, CHS), jnp.int32),
            pltpu.VMEM((2, CHS), jnp.int32),
            pltpu.VMEM((CHS, HID), jnp.float32),
            pltpu.VMEM((CHS, HID), jnp.float32),
            pltpu.VMEM((CHS,), jnp.int32),
            pltpu.VMEM((CHS,), jnp.int32),
            pltpu.SemaphoreType.DMA,
            pltpu.SemaphoreType.DMA,
            pltpu.SemaphoreType.DMA,
            pltpu.SemaphoreType.DMA,
            pltpu.SemaphoreType.DMA,
            pltpu.SemaphoreType.DMA,
        ],
    )(x, ei)


def _tc3_body(s0v, s1v, deg, xr, wl, bl, gsum_ref):
    agg = (s0v[0] + s1v[0]) / jnp.maximum(deg[...], 1.0)
    pre = jnp.dot(agg, wl[...], **_MM) + bl[...] + xr[...]
    sg = jnp.maximum(pre, 0.0)

    @pl.when(pl.program_id(0) == 0)
    def _():
        gsum_ref[...] = jnp.zeros_like(gsum_ref)

    gsum_ref[...] += jnp.sum(sg, axis=0, keepdims=True)


def _tc3(s, deg, xr, wl, bl):
    B = 1000
    g = NN // B
    row = lambda i: (i, 0)
    full = lambda i: (0, 0)
    return pl.pallas_call(
        _tc3_body,
        grid=(g,),
        in_specs=[
            pl.BlockSpec((1, B, HID), lambda i: (0, i, 0)),
            pl.BlockSpec((1, B, HID), lambda i: (1, i, 0)),
            pl.BlockSpec((B, 1), row),
            pl.BlockSpec((B, HID), row),
            pl.BlockSpec((HID, HID), full),
            pl.BlockSpec((1, HID), full),
        ],
        out_specs=pl.BlockSpec((1, HID), full),
        out_shape=jax.ShapeDtypeStruct((1, HID), jnp.float32),
    )(s, s, deg, xr, wl, bl)


def _tc4_body(so, gsum, out_ref):
    out_ref[...] = so[...] + gsum[...] * (1.0 / NN)


def _tc4(so, gsum):
    B = 1000
    g = NN // B
    row = lambda i: (i, 0)
    full = lambda i: (0, 0)
    return pl.pallas_call(
        _tc4_body,
        grid=(g,),
        in_specs=[
            pl.BlockSpec((B, HID), row),
            pl.BlockSpec((1, HID), full),
        ],
        out_specs=pl.BlockSpec((B, HID), row),
        out_shape=jax.ShapeDtypeStruct((NN, HID), jnp.float32),
    )(so, gsum)


def kernel(structural_features, node_features, edge_index, W1, b1, W2, b2,
           gat_W, att_src, att_dst, gat_b, sage_Wl, sage_bl, sage_Wr):
    rows = jnp.arange(HID, dtype=jnp.int32)
    heads = rows // OUT
    hmask = (heads[:, None] == jnp.arange(H)[None, :]).astype(jnp.float32)
    a_s = hmask * att_src.reshape(-1)[:, None]
    a_d = hmask * att_dst.reshape(-1)[:, None]
    rmat = hmask.T

    so, hxa, adst = _tc1(structural_features, node_features,
                         W1, b1.reshape(1, HID), W2, b2.reshape(1, HID),
                         gat_W, a_s, a_d)
    pn, pt = _gat_edges(hxa, adst, edge_index)
    x, xr, deg = _tc2(pn, pt, hxa, adst, gat_b.reshape(1, HID), rmat, sage_Wr)
    s = _sage_edges(x, edge_index)
    gsum = _tc3(s, deg, xr, sage_Wl, sage_bl.reshape(1, HID))
    return _tc4(so, gsum)


# speedup vs baseline: 1.2230x; 1.2230x over previous
import functools

import jax
import jax.numpy as jnp
from jax import lax
from jax.experimental import pallas as pl
from jax.experimental.pallas import tpu as pltpu
from jax.experimental.pallas import tpu_sc as plsc

NN = 10000
EE = 320000
H = 8
OUT = 16
HID = 128
HXA_W = 144

NTILES = 32
CHG = 64
CHS = 128
ZCH = 80
NZ = NN // ZCH

_MM = dict(preferred_element_type=jnp.float32)


def _tc1_body(sf, nf, w1, b1, w2, b2, gw, a_s, a_d, so_ref, hxa_ref, adst_ref):
    h1 = jnp.maximum(jnp.dot(sf[...], w1[...], **_MM) + b1[...], 0.0)
    so_ref[...] = jnp.maximum(jnp.dot(h1, w2[...], **_MM) + b2[...], 0.0)
    hx = jnp.dot(nf[...], gw[...], **_MM)
    hxa_ref[:, 0:128] = hx
    hxa_ref[:, 128:136] = jnp.dot(hx, a_s[...], **_MM)
    hxa_ref[:, 136:144] = jnp.zeros_like(hxa_ref[:, 136:144])
    adst_ref[:, 0:8] = jnp.dot(hx, a_d[...], **_MM)
    adst_ref[:, 8:16] = jnp.zeros_like(adst_ref[:, 8:16])


def _tc1(sf, nf, w1, b1, w2, b2, gw, a_s, a_d):
    B = 1000
    g = NN // B
    row = lambda i: (i, 0)
    full = lambda i: (0, 0)
    return pl.pallas_call(
        _tc1_body,
        grid=(g,),
        in_specs=[
            pl.BlockSpec((B, 65), row),
            pl.BlockSpec((B, HID), row),
            pl.BlockSpec((65, HID), full),
            pl.BlockSpec((1, HID), full),
            pl.BlockSpec((HID, HID), full),
            pl.BlockSpec((1, HID), full),
            pl.BlockSpec((HID, HID), full),
            pl.BlockSpec((HID, H), full),
            pl.BlockSpec((HID, H), full),
        ],
        out_specs=[
            pl.BlockSpec((B, HID), row),
            pl.BlockSpec((B, HXA_W), row),
            pl.BlockSpec((B, 16), row),
        ],
        out_shape=[
            jax.ShapeDtypeStruct((NN, HID), jnp.float32),
            jax.ShapeDtypeStruct((NN, HXA_W), jnp.float32),
            jax.ShapeDtypeStruct((NN, 16), jnp.float32),
        ],
    )(sf, nf, w1, b1, w2, b2, gw, a_s, a_d)


def _zero_accum(buf_v, accum, sid, width, zch):
    nz = NN // zch

    def zrow(r, _):
        for kk in range(width // 16):
            buf_v[r, pl.ds(kk * 16, 16)] = jnp.zeros((16,), jnp.float32)
        return 0

    lax.fori_loop(0, zch, zrow, 0)

    def zac(k, _):
        idx = sid + k * 16

        @pl.when(idx < nz)
        def _():
            pltpu.sync_copy(buf_v.at[pl.ds(0, zch)],
                            accum.at[pl.ds(idx * zch, zch)])

        return 0

    lax.fori_loop(0, (nz + 15) // 16, zac, 0)


def _writeback(accum, out_hbm, cid, sid):
    def wb(k, _):
        idx = sid + k * 16

        @pl.when(idx < NZ)
        def _():
            pltpu.sync_copy(accum.at[pl.ds(idx * ZCH, ZCH)],
                            out_hbm.at[cid, pl.ds(idx * ZCH, ZCH)])

        return 0

    lax.fori_loop(0, (NZ + 15) // 16, wb, 0)


def _writeback_split(accum, out1_hbm, out2_hbm, cid, sid):
    def wb(k, _):
        idx = sid + k * 16

        @pl.when(idx < NZ)
        def _():
            pltpu.sync_copy(accum.at[pl.ds(idx * ZCH, ZCH), pl.ds(0, HID)],
                            out1_hbm.at[cid, pl.ds(idx * ZCH, ZCH)])
            pltpu.sync_copy(accum.at[pl.ds(idx * ZCH, ZCH), pl.ds(HID, 16)],
                            out2_hbm.at[cid, pl.ds(idx * ZCH, ZCH)])

        return 0

    lax.fori_loop(0, (NZ + 15) // 16, wb, 0)


def _copy_dsc(ei_v, dsc_v, ch):
    for kk in range(ch // 16):
        dsc_v[pl.ds(kk * 16, 16)] = ei_v[1, pl.ds(kk * 16, 16)]


def _gat_edges_body(hxa_hbm, adst_hbm, ei_hbm, out1_hbm, out2_hbm, accum,
                    ei_v0, ei_v1, ei_v2, hxa_v0, hxa_v1, hxa_v2,
                    adst_v0, adst_v1, adst_v2, dsc_v0, dsc_v1, dsc_v2,
                    sem_i0, sem_i1, sem_i2, sem_g0, sem_g1, sem_g2,
                    sem_a0, sem_a1, sem_a2, sem_s0, sem_s1, sem_s2):
    cid = lax.axis_index("c")
    sid = lax.axis_index("s")
    wid = sid * 2 + cid
    nchunk = EE // CHG
    njs = (nchunk + NTILES - 1) // NTILES
    nj = (nchunk + NTILES - 1 - wid) // NTILES

    ei_v = (ei_v0, ei_v1, ei_v2)
    hxa_v = (hxa_v0, hxa_v1, hxa_v2)
    adst_v = (adst_v0, adst_v1, adst_v2)
    dsc_v = (dsc_v0, dsc_v1, dsc_v2)
    sem_i = (sem_i0, sem_i1, sem_i2)
    sem_g = (sem_g0, sem_g1, sem_g2)
    sem_a = (sem_a0, sem_a1, sem_a2)
    sem_s = (sem_s0, sem_s1, sem_s2)

    _zero_accum(hxa_v0, accum, sid, HXA_W, 40)
    plsc.subcore_barrier()

    def idx_args(j, s):
        return ei_hbm.at[:, pl.ds((wid + j * NTILES) * CHG, CHG)], ei_v[s], sem_i[s]

    def gather_args(s):
        return ((hxa_hbm.at[ei_v[s].at[0]], hxa_v[s], sem_g[s]),
                (adst_hbm.at[ei_v[s].at[1]], adst_v[s], sem_a[s]))

    def scatter_args(s):
        return hxa_v[s], accum.at[dsc_v[s]], sem_s[s]

    def compute(s):
        lane = lax.iota(jnp.int32, 16)
        mlo = jnp.where(lane < 8, 1.0, 0.0)
        dg1 = jnp.where(lane == 8, 1.0, 0.0)

        @plsc.parallel_loop(0, CHG, unroll=4)
        def _(c):
            av = hxa_v[s][c, pl.ds(128, 16)] + adst_v[s][c, pl.ds(0, 16)]
            al = jnp.where(av > 0.0, av, av * 0.2)
            tail = jnp.exp(al) * mlo + dg1
            for h in range(H):
                hxa_v[s][c, pl.ds(h * 16, 16)] = (
                    hxa_v[s][c, pl.ds(h * 16, 16)] * tail[h])
            hxa_v[s][c, pl.ds(128, 16)] = tail

    pltpu.async_copy(*idx_args(0, 0)).wait()
    for a in gather_args(0):
        pltpu.async_copy(*a)
    pltpu.async_copy(*idx_args(1, 1))
    pltpu.async_copy(*idx_args(2, 2))

    def pipe(jj, _):
        for b in (0, 1, 2):
            j = jj * 3 + b
            b1 = (b + 1) % 3

            @pl.when(j + 1 < nj)
            def _():
                pltpu.make_async_copy(*idx_args(j + 1, b1)).wait()

                @pl.when(j >= 2)
                def _():
                    pltpu.make_async_copy(*scatter_args(b1)).wait()

                for a in gather_args(b1):
                    pltpu.async_copy(*a)

            @pl.when(j < nj)
            def _():
                for a in gather_args(b):
                    pltpu.make_async_copy(*a).wait()
                _copy_dsc(ei_v[b], dsc_v[b], CHG)

            @pl.when(j + 3 < nj)
            def _():
                pltpu.async_copy(*idx_args(j + 3, b))

            @pl.when(j < nj)
            def _():
                compute(b)
                pltpu.async_copy(*scatter_args(b), add=True)

        return 0

    lax.fori_loop(0, (njs + 2) // 3, pipe, 0)
    pltpu.make_async_copy(*scatter_args(0)).wait()
    pltpu.make_async_copy(*scatter_args(1)).wait()
    pltpu.make_async_copy(*scatter_args(2)).wait()
    plsc.subcore_barrier()
    _writeback_split(accum, out1_hbm, out2_hbm, cid, sid)


def _gat_edges(hxa, adst, ei):
    mesh = plsc.VectorSubcoreMesh(core_axis_name="c", subcore_axis_name="s", num_cores=2, num_subcores=16)
    return pl.kernel(
        _gat_edges_body,
        out_type=[jax.ShapeDtypeStruct((2, NN, HID), jnp.float32),
                  jax.ShapeDtypeStruct((2, NN, 16), jnp.float32)],
        mesh=mesh,
        compiler_params=pltpu.CompilerParams(use_tc_tiling_on_sc=False),
        scratch_types=[
            pltpu.VMEM_SHARED((NN, HXA_W), jnp.float32),
            pltpu.VMEM((2, CHG), jnp.int32),
            pltpu.VMEM((2, CHG), jnp.int32),
            pltpu.VMEM((2, CHG), jnp.int32),
            pltpu.VMEM((CHG, HXA_W), jnp.float32),
            pltpu.VMEM((CHG, HXA_W), jnp.float32),
            pltpu.VMEM((CHG, HXA_W), jnp.float32),
            pltpu.VMEM((CHG, 16), jnp.float32),
            pltpu.VMEM((CHG, 16), jnp.float32),
            pltpu.VMEM((CHG, 16), jnp.float32),
            pltpu.VMEM((CHG,), jnp.int32),
            pltpu.VMEM((CHG,), jnp.int32),
            pltpu.VMEM((CHG,), jnp.int32),
            pltpu.SemaphoreType.DMA,
            pltpu.SemaphoreType.DMA,
            pltpu.SemaphoreType.DMA,
            pltpu.SemaphoreType.DMA,
            pltpu.SemaphoreType.DMA,
            pltpu.SemaphoreType.DMA,
            pltpu.SemaphoreType.DMA,
            pltpu.SemaphoreType.DMA,
            pltpu.SemaphoreType.DMA,
            pltpu.SemaphoreType.DMA,
            pltpu.SemaphoreType.DMA,
            pltpu.SemaphoreType.DMA,
        ],
    )(hxa, adst, ei)


def _tc2_body(pn0v, pn1v, pt0v, pt1v, hxa, adst, gb, rmat, wr,
              x_ref, xr_ref, deg_ref):
    pn0 = pn0v[0]
    pn1 = pn1v[0]
    pt0 = pt0v[0]
    pt1 = pt1v[0]
    a_s = hxa[:, 128:136]
    a_d = adst[:, 0:8]
    t = a_s + a_d
    al = jnp.where(t > 0.0, t, t * 0.2)
    exl = jnp.exp(al)
    denom = pt0[:, 0:8] + pt1[:, 0:8] + exl
    exl_rep = jnp.dot(exl, rmat[...], **_MM)
    den_rep = jnp.dot(denom, rmat[...], **_MM)
    msum = pn0[...] + pn1[...] + hxa[:, 0:128] * exl_rep
    gat = msum / (den_rep + 1e-16)
    xx = gat + gb[...]
    x = jnp.where(xx > 0.0, xx, xx * 0.2)
    x_ref[...] = x
    xr_ref[...] = jnp.dot(x, wr[...], **_MM)
    deg_ref[...] = pt0[:, 8:9] + pt1[:, 8:9]


def _tc2(pn, pt, hxa, adst, gb, rmat, wr):
    B = 1000
    g = NN // B
    row = lambda i: (i, 0)
    full = lambda i: (0, 0)
    return pl.pallas_call(
        _tc2_body,
        grid=(g,),
        in_specs=[
            pl.BlockSpec((1, B, HID), lambda i: (0, i, 0)),
            pl.BlockSpec((1, B, HID), lambda i: (1, i, 0)),
            pl.BlockSpec((1, B, 16), lambda i: (0, i, 0)),
            pl.BlockSpec((1, B, 16), lambda i: (1, i, 0)),
            pl.BlockSpec((B, HXA_W), row),
            pl.BlockSpec((B, 16), row),
            pl.BlockSpec((1, HID), full),
            pl.BlockSpec((H, HID), full),
            pl.BlockSpec((HID, HID), full),
        ],
        out_specs=[
            pl.BlockSpec((B, HID), row),
            pl.BlockSpec((B, HID), row),
            pl.BlockSpec((B, 1), row),
        ],
        out_shape=[
            jax.ShapeDtypeStruct((NN, HID), jnp.float32),
            jax.ShapeDtypeStruct((NN, HID), jnp.float32),
            jax.ShapeDtypeStruct((NN, 1), jnp.float32),
        ],
    )(pn, pn, pt, pt, hxa, adst, gb, rmat, wr)


def _sage_edges_body(x_hbm, ei_hbm, out_hbm, accum,
                     ei_v0, ei_v1, x_v0, x_v1, dsc_v0, dsc_v1,
                     sem_i0, sem_i1, sem_g0, sem_g1, sem_s0, sem_s1):
    cid = lax.axis_index("c")
    sid = lax.axis_index("s")
    wid = sid * 2 + cid
    nchunk = EE // CHS
    njs = (nchunk + NTILES - 1) // NTILES
    nj = (nchunk + NTILES - 1 - wid) // NTILES

    ei_v = (ei_v0, ei_v1)
    x_v = (x_v0, x_v1)
    dsc_v = (dsc_v0, dsc_v1)
    sem_i = (sem_i0, sem_i1)
    sem_g = (sem_g0, sem_g1)
    sem_s = (sem_s0, sem_s1)

    _zero_accum(x_v0, accum, sid, HID, ZCH)
    plsc.subcore_barrier()

    def idx_args(j, s):
        return ei_hbm.at[:, pl.ds((wid + j * NTILES) * CHS, CHS)], ei_v[s], sem_i[s]

    def gather_args(s):
        return x_hbm.at[ei_v[s].at[0]], x_v[s], sem_g[s]

    def scatter_args(s):
        return x_v[s], accum.at[dsc_v[s]], sem_s[s]

    pltpu.async_copy(*idx_args(0, 0)).wait()
    pltpu.async_copy(*gather_args(0))
    pltpu.async_copy(*idx_args(1, 1))

    def pipe(jj, _):
        for b in (0, 1):
            j = jj * 2 + b
            nb = 1 - b

            @pl.when(j + 1 < nj)
            def _():
                pltpu.make_async_copy(*idx_args(j + 1, nb)).wait()

                @pl.when(j >= 1)
                def _():
                    pltpu.make_async_copy(*scatter_args(nb)).wait()

                pltpu.async_copy(*gather_args(nb))

            @pl.when(j < nj)
            def _():
                pltpu.make_async_copy(*gather_args(b)).wait()
                _copy_dsc(ei_v[b], dsc_v[b], CHS)

            @pl.when(j + 2 < nj)
            def _():
                pltpu.async_copy(*idx_args(j + 2, b))

            @pl.when(j < nj)
            def _():
                pltpu.async_copy(*scatter_args(b), add=True)

        return 0

    lax.fori_loop(0, (njs + 1) // 2, pipe, 0)
    pltpu.make_async_copy(*scatter_args(0)).wait()
    pltpu.make_async_copy(*scatter_args(1)).wait()
    plsc.subcore_barrier()
    _writeback(accum, out_hbm, cid, sid)


def _sage_edges(x, ei):
    mesh = plsc.VectorSubcoreMesh(core_axis_name="c", subcore_axis_name="s", num_cores=2, num_subcores=16)
    return pl.kernel(
        _sage_edges_body,
        out_type=jax.ShapeDtypeStruct((2, NN, HID), jnp.float32),
        mesh=mesh,
        compiler_params=pltpu.CompilerParams(use_tc_tiling_on_sc=False),
        scratch_types=[
            pltpu.VMEM_SHARED((NN, HID), jnp.float32),
            pltpu.VMEM((2, CHS), jnp.int32),
            pltpu.VMEM((2, CHS), jnp.int32),
            pltpu.VMEM((CHS, HID), jnp.float32),
            pltpu.VMEM((CHS, HID), jnp.float32),
            pltpu.VMEM((CHS,), jnp.int32),
            pltpu.VMEM((CHS,), jnp.int32),
            pltpu.SemaphoreType.DMA,
            pltpu.SemaphoreType.DMA,
            pltpu.SemaphoreType.DMA,
            pltpu.SemaphoreType.DMA,
            pltpu.SemaphoreType.DMA,
            pltpu.SemaphoreType.DMA,
        ],
    )(x, ei)


def _tc3_body(s0v, s1v, deg, xr, wl, bl, gsum_ref):
    agg = (s0v[0] + s1v[0]) / jnp.maximum(deg[...], 1.0)
    pre = jnp.dot(agg, wl[...], **_MM) + bl[...] + xr[...]
    sg = jnp.maximum(pre, 0.0)

    @pl.when(pl.program_id(0) == 0)
    def _():
        gsum_ref[...] = jnp.zeros_like(gsum_ref)

    gsum_ref[...] += jnp.sum(sg, axis=0, keepdims=True)


def _tc3(s, deg, xr, wl, bl):
    B = 1000
    g = NN // B
    row = lambda i: (i, 0)
    full = lambda i: (0, 0)
    return pl.pallas_call(
        _tc3_body,
        grid=(g,),
        in_specs=[
            pl.BlockSpec((1, B, HID), lambda i: (0, i, 0)),
            pl.BlockSpec((1, B, HID), lambda i: (1, i, 0)),
            pl.BlockSpec((B, 1), row),
            pl.BlockSpec((B, HID), row),
            pl.BlockSpec((HID, HID), full),
            pl.BlockSpec((1, HID), full),
        ],
        out_specs=pl.BlockSpec((1, HID), full),
        out_shape=jax.ShapeDtypeStruct((1, HID), jnp.float32),
    )(s, s, deg, xr, wl, bl)


def _tc4_body(so, gsum, out_ref):
    out_ref[...] = so[...] + gsum[...] * (1.0 / NN)


def _tc4(so, gsum):
    B = 1000
    g = NN // B
    row = lambda i: (i, 0)
    full = lambda i: (0, 0)
    return pl.pallas_call(
        _tc4_body,
        grid=(g,),
        in_specs=[
            pl.BlockSpec((B, HID), row),
            pl.BlockSpec((1, HID), full),
        ],
        out_specs=pl.BlockSpec((B, HID), row),
        out_shape=jax.ShapeDtypeStruct((NN, HID), jnp.float32),
    )(so, gsum)


def kernel(structural_features, node_features, edge_index, W1, b1, W2, b2,
           gat_W, att_src, att_dst, gat_b, sage_Wl, sage_bl, sage_Wr):
    rows = jnp.arange(HID, dtype=jnp.int32)
    heads = rows // OUT
    hmask = (heads[:, None] == jnp.arange(H)[None, :]).astype(jnp.float32)
    a_s = hmask * att_src.reshape(-1)[:, None]
    a_d = hmask * att_dst.reshape(-1)[:, None]
    rmat = hmask.T

    so, hxa, adst = _tc1(structural_features, node_features,
                         W1, b1.reshape(1, HID), W2, b2.reshape(1, HID),
                         gat_W, a_s, a_d)
    pn, pt = _gat_edges(hxa, adst, edge_index)
    x, xr, deg = _tc2(pn, pt, hxa, adst, gat_b.reshape(1, HID), rmat, sage_Wr)
    s = _sage_edges(x, edge_index)
    gsum = _tc3(s, deg, xr, sage_Wl, sage_bl.reshape(1, HID))
    return _tc4(so, gsum)
